# baseline (speedup 1.0000x reference)
import functools

import jax
import jax.numpy as jnp
from jax import lax
from jax.experimental import pallas as pl
from jax.experimental.pallas import tpu as pltpu
from jax.experimental.pallas import tpu_sc as plsc

N = 100000
NODE_DIM = 512
HIDDEN_DIM = 256
NUM_SEGMENTS = 512
N_ELEMENTS = 119

NUM_WORKERS = 32
NODES_PER_TILE = 3136
NP = NUM_WORKERS * NODES_PER_TILE
VECS_PER_TILE = NODES_PER_TILE // 16
LANES = 16
NFEAT = 9
ACC_WORDS = NFEAT * LANES * NUM_SEGMENTS
RED_WORDS = NFEAT * NUM_SEGMENTS


def _mlp_body(x_ref, w1t_ref, b1_ref, w2_ref, b2_ref, out_ref):
    y = jnp.dot(x_ref[...], w1t_ref[...], preferred_element_type=jnp.float32)
    y = y + b1_ref[...]
    h = y * (1.0 / (1.0 + jnp.exp(-y)))
    s = lax.dot_general(w2_ref[...], h, (((1,), (1,)), ((), ())),
                        preferred_element_type=jnp.float32) + b2_ref[0, 0]
    cols = s.shape[1]
    gid = pl.program_id(0) * cols + lax.broadcasted_iota(jnp.int32, (1, cols), 1)
    s = s * (gid < N).astype(jnp.float32)
    out_ref[...] = s.reshape(-1)


def _mlp_scalar(x, w1t, b1_2d, w2, b2_2d, block_rows=7168):
    nblocks = NP // block_rows
    return pl.pallas_call(
        _mlp_body,
        grid=(nblocks,),
        in_specs=[
            pl.BlockSpec((block_rows, NODE_DIM), lambda i: (i, 0)),
            pl.BlockSpec((NODE_DIM, HIDDEN_DIM), lambda i: (0, 0)),
            pl.BlockSpec((1, HIDDEN_DIM), lambda i: (0, 0)),
            pl.BlockSpec((1, HIDDEN_DIM), lambda i: (0, 0)),
            pl.BlockSpec((1, 1), lambda i: (0, 0)),
        ],
        out_specs=pl.BlockSpec((block_rows,), lambda i: (i,)),
        out_shape=jax.ShapeDtypeStruct((NP,), jnp.float32),
    )(x, w1t, b1_2d, w2, b2_2d)


def _segsum_body(px_hbm, py_hbm, pz_hbm, s_hbm, an_hbm, masses_hbm, b_hbm,
                 out_hbm, px_v, py_v, pz_v, s_v, an_v, b_v, m128_v,
                 acc_v, red_v, sem):
    wid = lax.axis_index("s") * 2 + lax.axis_index("c")
    base = wid * NODES_PER_TILE

    copies = [pltpu.make_async_copy(h.at[pl.ds(base, NODES_PER_TILE)], v, sem)
              for h, v in ((px_hbm, px_v), (py_hbm, py_v), (pz_hbm, pz_v),
                           (s_hbm, s_v), (an_hbm, an_v), (b_hbm, b_v))]
    copies.append(pltpu.make_async_copy(masses_hbm, m128_v, sem))
    for c in copies:
        c.start()

    zeros16 = jnp.zeros((16,), jnp.float32)

    def zero_body(i, _):
        for u in range(16):
            acc_v[pl.ds(i * 256 + u * 16, 16)] = zeros16
        return 0

    lax.fori_loop(0, ACC_WORDS // 256, zero_body, 0)

    for c in copies:
        c.wait()

    lane = lax.iota(jnp.int32, 16)

    def body(j, _):
        for u in range(2):
            o = j * 32 + u * 16
            addr = lane * NUM_SEGMENTS + b_v[pl.ds(o, 16)]
            m = plsc.load_gather(m128_v, [an_v[pl.ds(o, 16)]])
            s = s_v[pl.ds(o, 16)]
            px = px_v[pl.ds(o, 16)]
            py = py_v[pl.ds(o, 16)]
            pz = pz_v[pl.ds(o, 16)]
            r2 = px * px + py * py + pz * pz
            feats = (m, m * px, m * py, m * pz,
                     s, s * px, s * py, s * pz, s * r2)
            for k, v in enumerate(feats):
                plsc.addupdate_scatter(
                    acc_v, [addr + (k * LANES * NUM_SEGMENTS)], v)
        return 0

    lax.fori_loop(0, VECS_PER_TILE // 2, body, 0)

    def lred_body(v, _):
        for k in range(NFEAT):
            o = k * LANES * NUM_SEGMENTS + v * 16
            acc = acc_v[pl.ds(o, 16)]
            for l in range(1, LANES):
                acc = acc + acc_v[pl.ds(o + l * NUM_SEGMENTS, 16)]
            red_v[pl.ds(k * NUM_SEGMENTS + v * 16, 16)] = acc
        return 0

    lax.fori_loop(0, NUM_SEGMENTS // 16, lred_body, 0)

    pltpu.sync_copy(red_v, out_hbm.at[pl.ds(wid * RED_WORDS, RED_WORDS)])


def _sc_segsum(px, py, pz, s_p, an_p, masses_p, batch_p):
    mesh = plsc.VectorSubcoreMesh(core_axis_name="c", subcore_axis_name="s")
    f = functools.partial(
        pl.kernel,
        mesh=mesh,
        compiler_params=pltpu.CompilerParams(needs_layout_passes=False),
        out_type=jax.ShapeDtypeStruct((NUM_WORKERS * RED_WORDS,), jnp.float32),
        scratch_types=(
            [pltpu.VMEM((NODES_PER_TILE,), jnp.float32) for _ in range(4)]
            + [pltpu.VMEM((NODES_PER_TILE,), jnp.int32),
               pltpu.VMEM((NODES_PER_TILE,), jnp.int32),
               pltpu.VMEM((128,), jnp.float32),
               pltpu.VMEM((ACC_WORDS,), jnp.float32),
               pltpu.VMEM((RED_WORDS,), jnp.float32),
               pltpu.SemaphoreType.DMA]),
    )(_segsum_body)
    return f(px, py, pz, s_p, an_p, masses_p, batch_p)


def _combine_body(p_ref, out_ref):
    t = jnp.sum(p_ref[...].reshape(NUM_WORKERS, NFEAT, NUM_SEGMENTS), axis=0)
    inv = 1.0 / t[0]
    cx = t[1] * inv
    cy = t[2] * inv
    cz = t[3] * inv
    res = t[8] - 2.0 * (cx * t[5] + cy * t[6] + cz * t[7]) \
        + (cx * cx + cy * cy + cz * cz) * t[4]
    out_ref[...] = res.reshape(1, NUM_SEGMENTS)


def _combine(partials_2d):
    return pl.pallas_call(
        _combine_body,
        in_specs=[pl.BlockSpec((NUM_WORKERS * NFEAT, NUM_SEGMENTS),
                               lambda: (0, 0))],
        out_specs=pl.BlockSpec((1, NUM_SEGMENTS), lambda: (0, 0)),
        out_shape=jax.ShapeDtypeStruct((1, NUM_SEGMENTS), jnp.float32),
    )(partials_2d)


def kernel(pos, node_invariant, batch, atomic_numbers, masses, W1, b1, W2, b2):
    pad = NP - N
    masses_p = jnp.pad(masses, (0, 128 - N_ELEMENTS))
    pos_t = jnp.pad(pos, ((0, pad), (0, 0))).T
    px, py, pz = pos_t[0], pos_t[1], pos_t[2]
    an_p = jnp.pad(atomic_numbers, (0, pad), constant_values=127)
    batch_p = jnp.pad(batch, (0, pad), constant_values=NUM_SEGMENTS - 1)

    s_p = _mlp_scalar(node_invariant, W1.T, b1.reshape(1, HIDDEN_DIM),
                      W2, b2.reshape(1, 1))
    partials = _sc_segsum(px, py, pz, s_p, an_p, masses_p, batch_p)
    out = _combine(partials.reshape(NUM_WORKERS * NFEAT, NUM_SEGMENTS))
    return out.reshape(NUM_SEGMENTS, 1)
